# Initial kernel scaffold; baseline (speedup 1.0000x reference)
import jax
import jax.numpy as jnp
from jax.experimental import pallas as pl


def kernel(x, rel0_src, rel0_dst, rel1_src, rel1_dst, W0_r0, W0_r1, Wl0, b0, W1_r0, W1_r1, Wl1, b1, W2_r0, W2_r1, Wl2, b2):
    raise NotImplementedError("write your pallas kernel here")


# speedup vs baseline: 1.0006x; 1.0006x over previous
import jax
import jax.numpy as jnp
from jax.experimental import pallas as pl

N = 10000


def _bias_relu_kernel(x_ref, b_ref, o_ref, *, act):
    v = x_ref[...] + b_ref[...]
    o_ref[...] = jnp.maximum(v, 0.0) if act else v


def _layer(h, edges, Wr0, Wr1, Wl, b, act):
    out = jnp.zeros((N, Wr0.shape[1]), dtype=h.dtype)
    for (src, dst), W in zip(edges, (Wr0, Wr1)):
        agg = jax.ops.segment_sum(h[src], dst, num_segments=N)
        deg = jax.ops.segment_sum(jnp.ones((src.shape[0],), dtype=h.dtype), dst, num_segments=N)
        deg = jnp.clip(deg, 1.0, None)
        out = out + (agg / deg[:, None]) @ W
    out = out + h @ Wl
    return pl.pallas_call(
        lambda x_ref, b_ref, o_ref: _bias_relu_kernel(x_ref, b_ref, o_ref, act=act),
        out_shape=jax.ShapeDtypeStruct(out.shape, out.dtype),
    )(out, jnp.broadcast_to(b, out.shape))


def kernel(x, rel0_src, rel0_dst, rel1_src, rel1_dst, W0_r0, W0_r1, Wl0, b0, W1_r0, W1_r1, Wl1, b1, W2_r0, W2_r1, Wl2, b2):
    edges = ((rel0_src, rel0_dst), (rel1_src, rel1_dst))
    h = _layer(x, edges, W0_r0, W0_r1, Wl0, b0, True)
    h = _layer(h, edges, W1_r0, W1_r1, Wl1, b1, True)
    h = _layer(h, edges, W2_r0, W2_r1, Wl2, b2, False)
    return h


# speedup vs baseline: 2.5916x; 2.5900x over previous
import functools

import jax
import jax.numpy as jnp
from jax import lax
from jax.experimental import pallas as pl
from jax.experimental.pallas import tpu as pltpu
from jax.experimental.pallas import tpu_sc as plsc

N = 10000
E = 160000
NSUB = 16
CHUNK = 128
NCHUNK = 80
EPT = NCHUNK * CHUNK
PADE = NSUB * EPT
ROWS_PER = 632
NROWS = NSUB * ROWS_PER
W64 = 64
MT = 400
GRID_M = N // MT

f32 = jnp.float32
_mesh = plsc.VectorSubcoreMesh(core_axis_name="c", subcore_axis_name="s")


@functools.partial(
    pl.kernel,
    out_type=(jax.ShapeDtypeStruct((NROWS, W64), f32),) * 4,
    mesh=_mesh,
    scratch_types=[
        pltpu.VMEM((NCHUNK, CHUNK), jnp.int32),
        pltpu.VMEM((NCHUNK, CHUNK), jnp.int32),
        pltpu.VMEM((CHUNK, W64), f32),
        pltpu.VMEM((CHUNK, W64), f32),
        pltpu.VMEM_SHARED((NROWS, W64), f32),
        pltpu.SemaphoreType.DMA,
        pltpu.SemaphoreType.DMA,
    ],
    compiler_params=pltpu.CompilerParams(use_tc_tiling_on_sc=False),
)
def _seg4(tblA0, tblA1, tblB0, tblB1, srcs, dsts, zrows,
          outA0, outA1, outB0, outB1,
          sidx, didx, bufa, bufb, acc, sema, semb):
    c = lax.axis_index("c")
    s = lax.axis_index("s")
    row0 = s * ROWS_PER
    pltpu.sync_copy(srcs.at[c, s], sidx)
    pltpu.sync_copy(dsts.at[c, s], didx)

    def run(tbl, out):
        pltpu.sync_copy(zrows, acc.at[pl.ds(row0, ROWS_PER)])
        plsc.subcore_barrier()
        pltpu.async_copy(tbl.at[sidx.at[0]], bufa, sema)

        def body(jj, carry):
            j = jj * 2
            pltpu.make_async_copy(tbl.at[sidx.at[j]], bufa, sema).wait()
            pltpu.async_copy(tbl.at[sidx.at[j + 1]], bufb, semb)
            pltpu.sync_copy(bufa, acc.at[didx.at[j]], add=True)
            pltpu.make_async_copy(tbl.at[sidx.at[j + 1]], bufb, semb).wait()

            @pl.when(jj < NCHUNK // 2 - 1)
            def _():
                pltpu.async_copy(tbl.at[sidx.at[j + 2]], bufa, sema)

            pltpu.sync_copy(bufb, acc.at[didx.at[j + 1]], add=True)
            return carry

        lax.fori_loop(0, NCHUNK // 2, body, 0)
        plsc.subcore_barrier()
        pltpu.sync_copy(acc.at[pl.ds(row0, ROWS_PER)],
                        out.at[pl.ds(row0, ROWS_PER)])

    @pl.when(c == 0)
    def _():
        run(tblA0, outA0)
        run(tblB0, outB0)

    @pl.when(c == 1)
    def _():
        run(tblA1, outA1)
        run(tblB1, outB1)


@functools.partial(
    pl.kernel,
    out_type=(jax.ShapeDtypeStruct((NROWS, 16), f32),
              jax.ShapeDtypeStruct((NROWS, 16), f32)),
    mesh=_mesh,
    scratch_types=[
        pltpu.VMEM((NCHUNK, CHUNK), jnp.int32),
        pltpu.VMEM((CHUNK, 16), f32),
        pltpu.VMEM_SHARED((NROWS, 16), f32),
    ],
    compiler_params=pltpu.CompilerParams(use_tc_tiling_on_sc=False),
)
def _deg2(dsts, ones_rows, zrows, out0, out1, didx, onesv, acc):
    c = lax.axis_index("c")
    s = lax.axis_index("s")
    row0 = s * ROWS_PER
    pltpu.sync_copy(dsts.at[c, s], didx)
    pltpu.sync_copy(ones_rows, onesv)
    pltpu.sync_copy(zrows, acc.at[pl.ds(row0, ROWS_PER)])
    plsc.subcore_barrier()

    def body(j, carry):
        pltpu.sync_copy(onesv, acc.at[didx.at[j]], add=True)
        return carry

    lax.fori_loop(0, NCHUNK, body, 0)
    plsc.subcore_barrier()

    @pl.when(c == 0)
    def _():
        pltpu.sync_copy(acc.at[pl.ds(row0, ROWS_PER)],
                        out0.at[pl.ds(row0, ROWS_PER)])

    @pl.when(c == 1)
    def _():
        pltpu.sync_copy(acc.at[pl.ds(row0, ROWS_PER)],
                        out1.at[pl.ds(row0, ROWS_PER)])


def _prep(src, dst):
    pad = PADE - E
    src = jnp.concatenate([src.astype(jnp.int32),
                           jnp.zeros((pad,), jnp.int32)])
    dst = jnp.concatenate([dst.astype(jnp.int32),
                           jnp.full((pad,), N, jnp.int32)])
    return src.reshape(NSUB, NCHUNK, CHUNK), dst.reshape(NSUB, NCHUNK, CHUNK)


def _dense_combine(mats, scaled_by, degs, Ws, b, act, out_widths):
    OUT = Ws[0].shape[1]
    nm = len(mats)
    nd = len(degs)

    def body(*refs):
        mrefs = refs[:nm]
        drefs = refs[nm:nm + nd]
        wrefs = refs[nm + nd:nm + nd + nm]
        bref = refs[nm + nd + nm]
        orefs = refs[nm + nd + nm + 1:]
        rs = [1.0 / jnp.maximum(dr[:, 0:1], 1.0) for dr in
              [d[...] for d in drefs]]
        res = jnp.zeros((MT, OUT), f32)
        for mref, sb, wref in zip(mrefs, scaled_by, wrefs):
            xm = mref[...]
            if sb is not None:
                xm = xm * rs[sb]
            res = res + jnp.dot(xm, wref[...], preferred_element_type=f32)
        res = res + bref[...]
        if act:
            res = jnp.maximum(res, 0.0)
        off = 0
        for oref, w in zip(orefs, out_widths):
            oref[...] = res[:, off:off + w]
            off += w

    in_specs = (
        [pl.BlockSpec((MT, m.shape[1]), lambda i: (i, 0)) for m in mats]
        + [pl.BlockSpec((MT, 16), lambda i: (i, 0)) for _ in degs]
        + [pl.BlockSpec(w.shape, lambda i: (0, 0)) for w in Ws]
        + [pl.BlockSpec((1, OUT), lambda i: (0, 0))]
    )
    out_shape = tuple(jax.ShapeDtypeStruct((N, w), f32) for w in out_widths)
    out_specs = tuple(pl.BlockSpec((MT, w), lambda i: (i, 0))
                      for w in out_widths)
    res = pl.pallas_call(
        body, grid=(GRID_M,), in_specs=in_specs, out_specs=out_specs,
        out_shape=out_shape,
    )(*mats, *degs, *Ws, b.reshape(1, OUT))
    return res


def _premm(hmats, W0s, W1s, Wls, b):
    nh = len(hmats)

    def body(*refs):
        hrefs = refs[:nh]
        w0refs = refs[nh:2 * nh]
        w1refs = refs[2 * nh:3 * nh]
        wlrefs = refs[3 * nh:4 * nh]
        bref = refs[4 * nh]
        o0a, o0b, o1a, o1b, o2 = refs[4 * nh + 1:]
        hs = [h[...] for h in hrefs]
        r0 = sum(jnp.dot(h, w[...], preferred_element_type=f32)
                 for h, w in zip(hs, w0refs))
        r1 = sum(jnp.dot(h, w[...], preferred_element_type=f32)
                 for h, w in zip(hs, w1refs))
        r2 = sum(jnp.dot(h, w[...], preferred_element_type=f32)
                 for h, w in zip(hs, wlrefs)) + bref[...]
        o0a[...] = r0[:, :W64]
        o0b[...] = r0[:, W64:]
        o1a[...] = r1[:, :W64]
        o1b[...] = r1[:, W64:]
        o2[...] = r2

    in_specs = (
        [pl.BlockSpec((MT, W64), lambda i: (i, 0)) for _ in hmats]
        + [pl.BlockSpec(w.shape, lambda i: (0, 0))
           for w in (*W0s, *W1s, *Wls)]
        + [pl.BlockSpec((1, 128), lambda i: (0, 0))]
    )
    out_shape = (tuple(jax.ShapeDtypeStruct((N, W64), f32) for _ in range(4))
                 + (jax.ShapeDtypeStruct((N, 128), f32),))
    out_specs = (tuple(pl.BlockSpec((MT, W64), lambda i: (i, 0))
                       for _ in range(4))
                 + (pl.BlockSpec((MT, 128), lambda i: (i, 0)),))
    return pl.pallas_call(
        body, grid=(GRID_M,), in_specs=in_specs, out_specs=out_specs,
        out_shape=out_shape,
    )(*hmats, *W0s, *W1s, *Wls, b.reshape(1, 128))


def _final(zpieces, deg0, deg1, y2):
    def body(z0a, z0b, z1a, z1b, d0r, d1r, y2r, o):
        r0 = 1.0 / jnp.maximum(d0r[:, 0:1], 1.0)
        r1 = 1.0 / jnp.maximum(d1r[:, 0:1], 1.0)
        z0 = jnp.concatenate([z0a[...], z0b[...]], axis=1)
        z1 = jnp.concatenate([z1a[...], z1b[...]], axis=1)
        o[...] = y2r[...] + z0 * r0 + z1 * r1

    in_specs = (
        [pl.BlockSpec((MT, W64), lambda i: (i, 0))] * 4
        + [pl.BlockSpec((MT, 16), lambda i: (i, 0))] * 2
        + [pl.BlockSpec((MT, 128), lambda i: (i, 0))]
    )
    return pl.pallas_call(
        body, grid=(GRID_M,), in_specs=in_specs,
        out_specs=pl.BlockSpec((MT, 128), lambda i: (i, 0)),
        out_shape=jax.ShapeDtypeStruct((N, 128), f32),
    )(*zpieces, deg0, deg1, y2)


def kernel(x, rel0_src, rel0_dst, rel1_src, rel1_dst,
           W0_r0, W0_r1, Wl0, b0, W1_r0, W1_r1, Wl1, b1,
           W2_r0, W2_r1, Wl2, b2):
    s0, d0 = _prep(rel0_src, rel0_dst)
    s1, d1 = _prep(rel1_src, rel1_dst)
    srcs = jnp.stack([s0, s1])
    dsts = jnp.stack([d0, d1])
    zrows = jnp.zeros((ROWS_PER, W64), f32)
    zrows16 = jnp.zeros((ROWS_PER, 16), f32)
    ones16 = jnp.ones((CHUNK, 16), f32)

    deg0, deg1 = _deg2(dsts, ones16, zrows16)

    xa, xb = x[:, :W64], x[:, W64:]
    a0a, a1a, a0b, a1b = _seg4(xa, xa, xb, xb, srcs, dsts, zrows)
    h1 = _dense_combine(
        [a0a, a0b, a1a, a1b, x], [0, 0, 1, 1, None], [deg0, deg1],
        [W0_r0[:W64], W0_r0[W64:], W0_r1[:W64], W0_r1[W64:], Wl0],
        b0, True, (W64,) * 4)
    h10, h11, h12, h13 = h1

    b0a, b1a, b0b, b1b = _seg4(h10, h10, h11, h11, srcs, dsts, zrows)
    b0c, b1c, b0d, b1d = _seg4(h12, h12, h13, h13, srcs, dsts, zrows)
    h2 = _dense_combine(
        [b0a, b0b, b0c, b0d, b1a, b1b, b1c, b1d, h10, h11, h12, h13],
        [0, 0, 0, 0, 1, 1, 1, 1, None, None, None, None],
        [deg0, deg1],
        [W1_r0[:W64], W1_r0[W64:128], W1_r0[128:192], W1_r0[192:],
         W1_r1[:W64], W1_r1[W64:128], W1_r1[128:192], W1_r1[192:],
         Wl1[:W64], Wl1[W64:128], Wl1[128:192], Wl1[192:]],
        b1, True, (W64,) * 4)
    h20, h21, h22, h23 = h2

    hmats = [h20, h21, h22, h23]
    wrows = [(0, W64), (W64, 128), (128, 192), (192, 256)]
    y0a, y0b, y1a, y1b, y2 = _premm(
        hmats,
        [W2_r0[a:bb] for a, bb in wrows],
        [W2_r1[a:bb] for a, bb in wrows],
        [Wl2[a:bb] for a, bb in wrows],
        b2)
    z0a, z1a, z0b, z1b = _seg4(y0a, y1a, y0b, y1b, srcs, dsts, zrows)
    return _final([z0a, z0b, z1a, z1b], deg0, deg1, y2)


# speedup vs baseline: 2.9189x; 1.1263x over previous
import functools

import jax
import jax.numpy as jnp
from jax import lax
from jax.experimental import pallas as pl
from jax.experimental.pallas import tpu as pltpu
from jax.experimental.pallas import tpu_sc as plsc

N = 10000
E = 160000
NSUB = 16
CHUNK = 128
NCHUNK = 80
EPT = NCHUNK * CHUNK
PADE = NSUB * EPT
ROWS_PER = 632
NROWS = NSUB * ROWS_PER
W64 = 64
GSZ = 4
MT = 400
GRID_M = N // MT

f32 = jnp.float32
_mesh = plsc.VectorSubcoreMesh(core_axis_name="c", subcore_axis_name="s")


@functools.partial(
    pl.kernel,
    out_type=(jax.ShapeDtypeStruct((NROWS, W64), f32),) * 2,
    mesh=_mesh,
    scratch_types=[
        pltpu.VMEM((NCHUNK, CHUNK), jnp.int32),
        pltpu.VMEM((NCHUNK, CHUNK), jnp.int32),
        pltpu.VMEM((2 * GSZ, CHUNK, W64), f32),
        pltpu.VMEM_SHARED((NROWS, W64), f32),
        pltpu.SemaphoreType.DMA,
        pltpu.SemaphoreType.DMA,
        pltpu.SemaphoreType.DMA,
        pltpu.SemaphoreType.DMA,
    ],
    compiler_params=pltpu.CompilerParams(use_tc_tiling_on_sc=False),
)
def _seg2(tbl0, tbl1, srcs, dsts, zrows, out0, out1,
          sidx, didx, bufs, acc, gsemA, gsemB, ssemA, ssemB):
    c = lax.axis_index("c")
    s = lax.axis_index("s")
    row0 = s * ROWS_PER
    pltpu.sync_copy(srcs.at[c, s], sidx)
    pltpu.sync_copy(dsts.at[c, s], didx)

    def run(tbl, out):
        sets = ((0, gsemA, ssemA), (GSZ, gsemB, ssemB))

        def fire(base, boff, gsem):
            for b in range(GSZ):
                pltpu.async_copy(tbl.at[sidx.at[base + b]],
                                 bufs.at[boff + b], gsem)

        def consume(base, boff, gsem, ssem):
            for b in range(GSZ):
                pltpu.make_async_copy(tbl.at[sidx.at[base + b]],
                                      bufs.at[boff + b], gsem).wait()
                pltpu.async_copy(bufs.at[boff + b],
                                 acc.at[didx.at[base + b]], ssem, add=True)
            for b in range(GSZ):
                pltpu.make_async_copy(bufs.at[boff + b],
                                      acc.at[didx.at[base + b]], ssem).wait()

        pltpu.sync_copy(zrows, acc.at[pl.ds(row0, ROWS_PER)])
        plsc.subcore_barrier()
        fire(0, 0, gsemA)
        fire(GSZ, GSZ, gsemB)

        def body(gg, carry):
            k0 = gg * 2 * GSZ
            for half in range(2):
                base = k0 + half * GSZ
                boff, gsem, ssem = sets[half]
                consume(base, boff, gsem, ssem)

                @pl.when(base + 2 * GSZ < NCHUNK)
                def _():
                    fire(base + 2 * GSZ, boff, gsem)
            return carry

        lax.fori_loop(0, NCHUNK // (2 * GSZ), body, 0)
        plsc.subcore_barrier()
        pltpu.sync_copy(acc.at[pl.ds(row0, ROWS_PER)],
                        out.at[pl.ds(row0, ROWS_PER)])

    @pl.when(c == 0)
    def _():
        run(tbl0, out0)

    @pl.when(c == 1)
    def _():
        run(tbl1, out1)


@functools.partial(
    pl.kernel,
    out_type=(jax.ShapeDtypeStruct((NROWS, 16), f32),
              jax.ShapeDtypeStruct((NROWS, 16), f32)),
    mesh=_mesh,
    scratch_types=[
        pltpu.VMEM((NCHUNK, CHUNK), jnp.int32),
        pltpu.VMEM((CHUNK, 16), f32),
        pltpu.VMEM_SHARED((NROWS, 16), f32),
    ],
    compiler_params=pltpu.CompilerParams(use_tc_tiling_on_sc=False),
)
def _deg2(dsts, ones_rows, zrows, out0, out1, didx, onesv, acc):
    c = lax.axis_index("c")
    s = lax.axis_index("s")
    row0 = s * ROWS_PER
    pltpu.sync_copy(dsts.at[c, s], didx)
    pltpu.sync_copy(ones_rows, onesv)
    pltpu.sync_copy(zrows, acc.at[pl.ds(row0, ROWS_PER)])
    plsc.subcore_barrier()

    def body(j, carry):
        pltpu.sync_copy(onesv, acc.at[didx.at[j]], add=True)
        return carry

    lax.fori_loop(0, NCHUNK, body, 0)
    plsc.subcore_barrier()

    @pl.when(c == 0)
    def _():
        pltpu.sync_copy(acc.at[pl.ds(row0, ROWS_PER)],
                        out0.at[pl.ds(row0, ROWS_PER)])

    @pl.when(c == 1)
    def _():
        pltpu.sync_copy(acc.at[pl.ds(row0, ROWS_PER)],
                        out1.at[pl.ds(row0, ROWS_PER)])


def _prep(src, dst):
    pad = PADE - E
    src = jnp.concatenate([src.astype(jnp.int32),
                           jnp.zeros((pad,), jnp.int32)])
    dst = jnp.concatenate([dst.astype(jnp.int32),
                           jnp.full((pad,), N, jnp.int32)])
    return src.reshape(NSUB, NCHUNK, CHUNK), dst.reshape(NSUB, NCHUNK, CHUNK)


def _dense_combine(mats, scaled_by, degs, Ws, b, act, out_widths):
    OUT = Ws[0].shape[1]
    nm = len(mats)
    nd = len(degs)

    def body(*refs):
        mrefs = refs[:nm]
        drefs = refs[nm:nm + nd]
        wrefs = refs[nm + nd:nm + nd + nm]
        bref = refs[nm + nd + nm]
        orefs = refs[nm + nd + nm + 1:]
        rs = [1.0 / jnp.maximum(dr[:, 0:1], 1.0) for dr in
              [d[...] for d in drefs]]
        res = jnp.zeros((MT, OUT), f32)
        for mref, sb, wref in zip(mrefs, scaled_by, wrefs):
            xm = mref[...]
            if sb is not None:
                xm = xm * rs[sb]
            res = res + jnp.dot(xm, wref[...], preferred_element_type=f32)
        res = res + bref[...]
        if act:
            res = jnp.maximum(res, 0.0)
        off = 0
        for oref, w in zip(orefs, out_widths):
            oref[...] = res[:, off:off + w]
            off += w

    in_specs = (
        [pl.BlockSpec((MT, m.shape[1]), lambda i: (i, 0)) for m in mats]
        + [pl.BlockSpec((MT, 16), lambda i: (i, 0)) for _ in degs]
        + [pl.BlockSpec(w.shape, lambda i: (0, 0)) for w in Ws]
        + [pl.BlockSpec((1, OUT), lambda i: (0, 0))]
    )
    out_shape = tuple(jax.ShapeDtypeStruct((N, w), f32) for w in out_widths)
    out_specs = tuple(pl.BlockSpec((MT, w), lambda i: (i, 0))
                      for w in out_widths)
    res = pl.pallas_call(
        body, grid=(GRID_M,), in_specs=in_specs, out_specs=out_specs,
        out_shape=out_shape,
    )(*mats, *degs, *Ws, b.reshape(1, OUT))
    return res


def _premm(hmats, W0s, W1s, Wls, b):
    nh = len(hmats)

    def body(*refs):
        hrefs = refs[:nh]
        w0refs = refs[nh:2 * nh]
        w1refs = refs[2 * nh:3 * nh]
        wlrefs = refs[3 * nh:4 * nh]
        bref = refs[4 * nh]
        o0a, o0b, o1a, o1b, o2 = refs[4 * nh + 1:]
        hs = [h[...] for h in hrefs]
        r0 = sum(jnp.dot(h, w[...], preferred_element_type=f32)
                 for h, w in zip(hs, w0refs))
        r1 = sum(jnp.dot(h, w[...], preferred_element_type=f32)
                 for h, w in zip(hs, w1refs))
        r2 = sum(jnp.dot(h, w[...], preferred_element_type=f32)
                 for h, w in zip(hs, wlrefs)) + bref[...]
        o0a[...] = r0[:, :W64]
        o0b[...] = r0[:, W64:]
        o1a[...] = r1[:, :W64]
        o1b[...] = r1[:, W64:]
        o2[...] = r2

    in_specs = (
        [pl.BlockSpec((MT, W64), lambda i: (i, 0)) for _ in hmats]
        + [pl.BlockSpec(w.shape, lambda i: (0, 0))
           for w in (*W0s, *W1s, *Wls)]
        + [pl.BlockSpec((1, 128), lambda i: (0, 0))]
    )
    out_shape = (tuple(jax.ShapeDtypeStruct((N, W64), f32) for _ in range(4))
                 + (jax.ShapeDtypeStruct((N, 128), f32),))
    out_specs = (tuple(pl.BlockSpec((MT, W64), lambda i: (i, 0))
                       for _ in range(4))
                 + (pl.BlockSpec((MT, 128), lambda i: (i, 0)),))
    return pl.pallas_call(
        body, grid=(GRID_M,), in_specs=in_specs, out_specs=out_specs,
        out_shape=out_shape,
    )(*hmats, *W0s, *W1s, *Wls, b.reshape(1, 128))


def _final(zpieces, deg0, deg1, y2):
    def body(z0a, z0b, z1a, z1b, d0r, d1r, y2r, o):
        r0 = 1.0 / jnp.maximum(d0r[:, 0:1], 1.0)
        r1 = 1.0 / jnp.maximum(d1r[:, 0:1], 1.0)
        z0 = jnp.concatenate([z0a[...], z0b[...]], axis=1)
        z1 = jnp.concatenate([z1a[...], z1b[...]], axis=1)
        o[...] = y2r[...] + z0 * r0 + z1 * r1

    in_specs = (
        [pl.BlockSpec((MT, W64), lambda i: (i, 0))] * 4
        + [pl.BlockSpec((MT, 16), lambda i: (i, 0))] * 2
        + [pl.BlockSpec((MT, 128), lambda i: (i, 0))]
    )
    return pl.pallas_call(
        body, grid=(GRID_M,), in_specs=in_specs,
        out_specs=pl.BlockSpec((MT, 128), lambda i: (i, 0)),
        out_shape=jax.ShapeDtypeStruct((N, 128), f32),
    )(*zpieces, deg0, deg1, y2)


def kernel(x, rel0_src, rel0_dst, rel1_src, rel1_dst,
           W0_r0, W0_r1, Wl0, b0, W1_r0, W1_r1, Wl1, b1,
           W2_r0, W2_r1, Wl2, b2):
    s0, d0 = _prep(rel0_src, rel0_dst)
    s1, d1 = _prep(rel1_src, rel1_dst)
    srcs = jnp.stack([s0, s1])
    dsts = jnp.stack([d0, d1])
    zrows = jnp.zeros((ROWS_PER, W64), f32)
    zrows16 = jnp.zeros((ROWS_PER, 16), f32)
    ones16 = jnp.ones((CHUNK, 16), f32)

    deg0, deg1 = _deg2(dsts, ones16, zrows16)

    xa, xb = x[:, :W64], x[:, W64:]
    a0a, a1a = _seg2(xa, xa, srcs, dsts, zrows)
    a0b, a1b = _seg2(xb, xb, srcs, dsts, zrows)
    h1 = _dense_combine(
        [a0a, a0b, a1a, a1b, x], [0, 0, 1, 1, None], [deg0, deg1],
        [W0_r0[:W64], W0_r0[W64:], W0_r1[:W64], W0_r1[W64:], Wl0],
        b0, True, (W64,) * 4)
    h10, h11, h12, h13 = h1

    b0a, b1a = _seg2(h10, h10, srcs, dsts, zrows)
    b0b, b1b = _seg2(h11, h11, srcs, dsts, zrows)
    b0c, b1c = _seg2(h12, h12, srcs, dsts, zrows)
    b0d, b1d = _seg2(h13, h13, srcs, dsts, zrows)
    h2 = _dense_combine(
        [b0a, b0b, b0c, b0d, b1a, b1b, b1c, b1d, h10, h11, h12, h13],
        [0, 0, 0, 0, 1, 1, 1, 1, None, None, None, None],
        [deg0, deg1],
        [W1_r0[:W64], W1_r0[W64:128], W1_r0[128:192], W1_r0[192:],
         W1_r1[:W64], W1_r1[W64:128], W1_r1[128:192], W1_r1[192:],
         Wl1[:W64], Wl1[W64:128], Wl1[128:192], Wl1[192:]],
        b1, True, (W64,) * 4)
    h20, h21, h22, h23 = h2

    hmats = [h20, h21, h22, h23]
    wrows = [(0, W64), (W64, 128), (128, 192), (192, 256)]
    y0a, y0b, y1a, y1b, y2 = _premm(
        hmats,
        [W2_r0[a:bb] for a, bb in wrows],
        [W2_r1[a:bb] for a, bb in wrows],
        [Wl2[a:bb] for a, bb in wrows],
        b2)
    z0a, z1a = _seg2(y0a, y1a, srcs, dsts, zrows)
    z0b, z1b = _seg2(y0b, y1b, srcs, dsts, zrows)
    return _final([z0a, z0b, z1a, z1b], deg0, deg1, y2)
